# baseline (speedup 1.0000x reference)
import jax
import jax.numpy as jnp
from jax.experimental import pallas as pl


def _body(col_ref, row_ref, out_ref):
    h = out_ref.shape[2]
    w = out_ref.shape[3]
    d2 = col_ref.shape[1]
    col_t = col_ref[...].T
    row_t = row_ref[...].T
    x_part = jnp.broadcast_to(col_t[:, None, :], (d2, h, w))
    y_part = jnp.broadcast_to(row_t[:, :, None], (d2, h, w))
    out_ref[0] = jnp.concatenate([x_part, y_part], axis=0)


def kernel(tensor, row_embed, col_embed):
    b = tensor.shape[0]
    h, w = tensor.shape[-2], tensor.shape[-1]
    d2 = row_embed.shape[-1]
    d = 2 * d2
    return pl.pallas_call(
        _body,
        grid=(b,),
        in_specs=[
            pl.BlockSpec((w, d2), lambda i: (0, 0)),
            pl.BlockSpec((h, d2), lambda i: (0, 0)),
        ],
        out_specs=pl.BlockSpec((1, d, h, w), lambda i: (i, 0, 0, 0)),
        out_shape=jax.ShapeDtypeStruct((b, d, h, w), jnp.float32),
    )(col_embed, row_embed)


# speedup vs baseline: 2.4766x; 2.4766x over previous
import jax
import jax.numpy as jnp
from jax import lax
from jax.experimental import pallas as pl


def _body(col_ref, row_ref, out_ref):
    w, d2 = col_ref.shape
    h = row_ref.shape[0]
    hw = h * w
    lane = lax.broadcasted_iota(jnp.int32, (w, hw), 1)
    sub = lax.broadcasted_iota(jnp.int32, (w, hw), 0)
    sel_col = (lane % w == sub).astype(jnp.float32)
    sel_row = (lane // w == sub).astype(jnp.float32)
    dn = (((0,), (0,)), ((), ()))
    a = lax.dot_general(col_ref[...], sel_col, dn,
                        preferred_element_type=jnp.float32)
    b = lax.dot_general(row_ref[...], sel_row, dn,
                        preferred_element_type=jnp.float32)
    out_ref[0] = jnp.concatenate([a, b], axis=0)


def kernel(tensor, row_embed, col_embed):
    b = tensor.shape[0]
    h, w = tensor.shape[-2], tensor.shape[-1]
    d2 = row_embed.shape[-1]
    d = 2 * d2
    out = pl.pallas_call(
        _body,
        grid=(b,),
        in_specs=[
            pl.BlockSpec((w, d2), lambda i: (0, 0)),
            pl.BlockSpec((h, d2), lambda i: (0, 0)),
        ],
        out_specs=pl.BlockSpec((1, d, h * w), lambda i: (i, 0, 0)),
        out_shape=jax.ShapeDtypeStruct((b, d, h * w), jnp.float32),
    )(col_embed, row_embed)
    return out.reshape(b, d, h, w)


# speedup vs baseline: 2.6131x; 1.0551x over previous
import jax
import jax.numpy as jnp
from jax import lax
from jax.experimental import pallas as pl
from jax.experimental.pallas import tpu as pltpu


def _make_body(b):
    def _body(col_ref, row_ref, out_ref, pat_ref, sem_ref):
        w, d2 = col_ref.shape
        h = row_ref.shape[0]
        hw = h * w
        lane = lax.broadcasted_iota(jnp.int32, (w, hw), 1)
        sub = lax.broadcasted_iota(jnp.int32, (w, hw), 0)
        sel_col = (lane % w == sub).astype(jnp.float32)
        sel_row = (lane // w == sub).astype(jnp.float32)
        dn = (((0,), (0,)), ((), ()))
        a = lax.dot_general(col_ref[...], sel_col, dn,
                            preferred_element_type=jnp.float32,
                            precision=lax.Precision.HIGHEST)
        bb = lax.dot_general(row_ref[...], sel_row, dn,
                             preferred_element_type=jnp.float32,
                             precision=lax.Precision.HIGHEST)
        pat_ref[...] = jnp.concatenate([a, bb], axis=0)
        copies = [
            pltpu.make_async_copy(pat_ref, out_ref.at[i], sem_ref.at[i])
            for i in range(b)
        ]
        for c in copies:
            c.start()
        for c in copies:
            c.wait()
    return _body


def kernel(tensor, row_embed, col_embed):
    b = tensor.shape[0]
    h, w = tensor.shape[-2], tensor.shape[-1]
    d2 = row_embed.shape[-1]
    d = 2 * d2
    out = pl.pallas_call(
        _make_body(b),
        in_specs=[
            pl.BlockSpec(memory_space=pltpu.MemorySpace.VMEM),
            pl.BlockSpec(memory_space=pltpu.MemorySpace.VMEM),
        ],
        out_specs=pl.BlockSpec(memory_space=pl.ANY),
        out_shape=jax.ShapeDtypeStruct((b, d, h * w), jnp.float32),
        scratch_shapes=[
            pltpu.VMEM((d, h * w), jnp.float32),
            pltpu.SemaphoreType.DMA((b,)),
        ],
    )(col_embed[:w], row_embed[:h])
    return out.reshape(b, d, h, w)
